# baseline (speedup 1.0000x reference)
import jax
import jax.numpy as jnp
from jax.experimental import pallas as pl
from jax.experimental.pallas import tpu as pltpu

N1, N2 = 25, 10
DIN = 128
H0, H1 = 256, 128
P = 1 + N1 + N1 * N2
BB = 64


def _act(x):
    return jnp.where(x >= 0, x, 0.01 * x)


def _tower(f_ref, w1_ref, b1_ref, w2_ref, b2_ref):
    f = f_ref[...]
    h0 = f[:, 0, :]
    h1 = f[:, 1:1 + N1, :]
    h2 = f[:, 1 + N1:, :].reshape(BB, N1, N2, DIN)
    neigh0 = jnp.mean(h1, axis=1)
    neigh1 = jnp.mean(h2, axis=2)
    w1 = w1_ref[...]
    w1a, w1b = w1[:DIN], w1[DIN:]
    b1 = b1_ref[...]
    h0n = _act(jnp.dot(h0, w1a, preferred_element_type=jnp.float32)
               + jnp.dot(neigh0, w1b, preferred_element_type=jnp.float32)
               + b1)
    h1r = h1.reshape(BB * N1, DIN)
    n1r = neigh1.reshape(BB * N1, DIN)
    h1n = _act(jnp.dot(h1r, w1a, preferred_element_type=jnp.float32)
               + jnp.dot(n1r, w1b, preferred_element_type=jnp.float32)
               + b1)
    neigh = jnp.mean(h1n.reshape(BB, N1, H0), axis=1)
    w2 = w2_ref[...]
    w2a, w2b = w2[:H0], w2[H0:]
    b2 = b2_ref[...]
    h0f = _act(jnp.dot(h0n, w2a, preferred_element_type=jnp.float32)
               + jnp.dot(neigh, w2b, preferred_element_type=jnp.float32)
               + b2)
    return _act(h0f)


def _fused_kernel(uf_ref, if_ref, w1u_ref, b1u_ref, w2u_ref, b2u_ref,
                  w1i_ref, b1i_ref, w2i_ref, b2i_ref, wl_ref, bl_ref,
                  out_ref):
    uh = _tower(uf_ref, w1u_ref, b1u_ref, w2u_ref, b2u_ref)
    ih = _tower(if_ref, w1i_ref, b1i_ref, w2i_ref, b2i_ref)
    pred = jnp.dot(uh * ih, wl_ref[...],
                   preferred_element_type=jnp.float32) + bl_ref[...]
    out_ref[...] = jax.nn.sigmoid(pred)


def kernel(sampling_user_feat, sampling_item_feat, W1_u, b1_u, W2_u, b2_u,
           W1_i, b1_i, W2_i, b2_i, W_lin, b_lin):
    b = sampling_user_feat.shape[0]
    grid = (b // BB,)
    feat_spec = pl.BlockSpec((BB, P, DIN), lambda i: (i, 0, 0))
    w1_spec = pl.BlockSpec((2 * DIN, H0), lambda i: (0, 0))
    b1_spec = pl.BlockSpec((1, H0), lambda i: (0, 0))
    w2_spec = pl.BlockSpec((2 * H0, H1), lambda i: (0, 0))
    b2_spec = pl.BlockSpec((1, H1), lambda i: (0, 0))
    wl_spec = pl.BlockSpec((H1, 2), lambda i: (0, 0))
    bl_spec = pl.BlockSpec((1, 2), lambda i: (0, 0))
    out = pl.pallas_call(
        _fused_kernel,
        grid=grid,
        in_specs=[feat_spec, feat_spec,
                  w1_spec, b1_spec, w2_spec, b2_spec,
                  w1_spec, b1_spec, w2_spec, b2_spec,
                  wl_spec, bl_spec],
        out_specs=pl.BlockSpec((BB, 2), lambda i: (i, 0)),
        out_shape=jax.ShapeDtypeStruct((b, 2), jnp.float32),
        compiler_params=pltpu.CompilerParams(
            dimension_semantics=("parallel",)),
    )(sampling_user_feat, sampling_item_feat,
      W1_u, b1_u.reshape(1, H0), W2_u, b2_u.reshape(1, H1),
      W1_i, b1_i.reshape(1, H0), W2_i, b2_i.reshape(1, H1),
      W_lin, b_lin.reshape(1, 2))
    return out
